# baseline (speedup 1.0000x reference)
import jax
import jax.numpy as jnp
from jax import lax
from jax.experimental import pallas as pl
from jax.experimental.pallas import tpu as pltpu
from jax.experimental.pallas import tpu_sc as plsc


_NC = 2
_NS = 16
_NW = _NC * _NS
_K = 128
_GRP = 28


def _make_edge_agg(n_pad, d, gpw):
    rt = n_pad // _NS
    spw = gpw // 2
    ngrp = gpw // _GRP
    sgrp = _GRP // 2
    mesh = plsc.VectorSubcoreMesh(core_axis_name="c", subcore_axis_name="s")

    def body(table, srcw, dstw2, zeros, out,
             src_v, dst_v, rows0, rows1, acc, ga0, ga1, sa0, sa1):
        cid = lax.axis_index("c")
        sid = lax.axis_index("s")
        w = cid * _NS + sid
        pltpu.sync_copy(zeros, acc.at[pl.ds(sid * rt, rt)])
        plsc.subcore_barrier()

        def gather(u, buf, sem):
            a = pltpu.async_copy(table.at[src_v.at[2 * u]],
                                 buf.at[pl.ds(0, _K)], sem)
            b = pltpu.async_copy(table.at[src_v.at[2 * u + 1]],
                                 buf.at[pl.ds(_K, _K)], sem)
            return a, b

        def scat(u, buf, sem):
            return pltpu.async_copy(buf, acc.at[dst_v.at[u]], sem, add=True)

        def group(g, carry):
            pltpu.sync_copy(srcw.at[pl.ds(w * gpw + g * _GRP, _GRP)], src_v)
            pltpu.sync_copy(dstw2.at[pl.ds(w * spw + g * sgrp, sgrp)], dst_v)
            a0, a1 = gather(0, rows0, ga0)
            b0, b1 = gather(1, rows1, ga1)

            def step(i, c2):
                u = 2 * i
                a0.wait()
                a1.wait()
                sA = scat(u, rows0, sa0)
                b0.wait()
                b1.wait()
                sB = scat(u + 1, rows1, sa1)
                sA.wait()
                gather(u + 2, rows0, ga0)
                sB.wait()
                gather(u + 3, rows1, ga1)
                return c2

            lax.fori_loop(0, sgrp // 2 - 1, step, 0)
            a0.wait()
            a1.wait()
            sA = scat(sgrp - 2, rows0, sa0)
            b0.wait()
            b1.wait()
            sB = scat(sgrp - 1, rows1, sa1)
            sA.wait()
            sB.wait()
            return carry

        lax.fori_loop(0, ngrp, group, 0)
        plsc.subcore_barrier()
        pltpu.sync_copy(acc.at[pl.ds(sid * rt, rt)],
                        out.at[cid, pl.ds(sid * rt, rt)])

    return pl.kernel(
        body,
        mesh=mesh,
        out_type=jax.ShapeDtypeStruct((_NC, n_pad, d), jnp.float32),
        compiler_params=pltpu.CompilerParams(use_tc_tiling_on_sc=False),
        scratch_types=[
            pltpu.VMEM((_GRP, _K), jnp.int32),
            pltpu.VMEM((sgrp, 2 * _K), jnp.int32),
            pltpu.VMEM((2 * _K, d), jnp.float32),
            pltpu.VMEM((2 * _K, d), jnp.float32),
            pltpu.VMEM_SHARED((n_pad, d), jnp.float32),
            pltpu.SemaphoreType.DMA,
            pltpu.SemaphoreType.DMA,
            pltpu.SemaphoreType.DMA,
            pltpu.SemaphoreType.DMA,
        ],
    )


def _make_degree(n_pad, d, gpw):
    rt = n_pad // _NS
    spw = gpw // 2
    ngrp = gpw // _GRP
    sgrp = _GRP // 2
    mesh = plsc.VectorSubcoreMesh(core_axis_name="c", subcore_axis_name="s")

    def body(dstw2, zeros, ones, out, dst_v, ones_v, acc, d0, d1):
        cid = lax.axis_index("c")
        sid = lax.axis_index("s")
        w = cid * _NS + sid
        pltpu.sync_copy(zeros, acc.at[pl.ds(sid * rt, rt)])
        pltpu.sync_copy(ones, ones_v)
        plsc.subcore_barrier()

        def group(g, carry):
            pltpu.sync_copy(dstw2.at[pl.ds(w * spw + g * sgrp, sgrp)], dst_v)
            sA = pltpu.async_copy(ones_v, acc.at[dst_v.at[0]], d0, add=True)
            sB = pltpu.async_copy(ones_v, acc.at[dst_v.at[1]], d1, add=True)

            def step(i, c2):
                u = 2 * i
                sA.wait()
                pltpu.async_copy(ones_v, acc.at[dst_v.at[u + 2]], d0, add=True)
                sB.wait()
                pltpu.async_copy(ones_v, acc.at[dst_v.at[u + 3]], d1, add=True)
                return c2

            lax.fori_loop(0, sgrp // 2 - 1, step, 0)
            sA.wait()
            sB.wait()
            return carry

        lax.fori_loop(0, ngrp, group, 0)
        plsc.subcore_barrier()
        pltpu.sync_copy(acc.at[pl.ds(sid * rt, rt)],
                        out.at[cid, pl.ds(sid * rt, rt)])

    return pl.kernel(
        body,
        mesh=mesh,
        out_type=jax.ShapeDtypeStruct((_NC, n_pad, d), jnp.float32),
        compiler_params=pltpu.CompilerParams(use_tc_tiling_on_sc=False),
        scratch_types=[
            pltpu.VMEM((sgrp, 2 * _K), jnp.int32),
            pltpu.VMEM((2 * _K, d), jnp.float32),
            pltpu.VMEM_SHARED((n_pad, d), jnp.float32),
            pltpu.SemaphoreType.DMA,
            pltpu.SemaphoreType.DMA,
        ],
    )


def _tc1_body(x4_ref, w1blk_ref, dv4_ref, g1_ref):
    dinv4 = lax.rsqrt(1.0 + dv4_ref[0] + dv4_ref[1])
    h4 = jnp.dot(x4_ref[...], w1blk_ref[...],
                 preferred_element_type=jnp.float32)
    g1_ref[...] = dinv4 * h4


def _tc2_body(dv4_ref, p_ref, g1_ref, b1t4_ref, w2big_ref, g2_ref):
    dinv4 = lax.rsqrt(1.0 + dv4_ref[0] + dv4_ref[1])
    agg = p_ref[0] + p_ref[1] + g1_ref[...]
    out1 = jnp.maximum(dinv4 * agg + b1t4_ref[...], 0.0)
    g2_ref[...] = jnp.dot(dinv4 * out1, w2big_ref[...],
                          preferred_element_type=jnp.float32)


def _tc3_body(dv4_ref, p_ref, g2_ref, b2t4_ref, o_ref):
    r4 = g2_ref.shape[0]
    dinv4 = lax.rsqrt(1.0 + dv4_ref[0] + dv4_ref[1])
    z = dinv4 * (p_ref[0] + p_ref[1] + g2_ref[...]) + b2t4_ref[...]
    lane = lax.broadcasted_iota(jnp.int32, (r4, 128), 1)
    other = jnp.where(lane % 32 == 0, jnp.roll(z, -1, axis=1),
                      jnp.roll(z, 1, axis=1))
    m = jnp.maximum(z, other)
    o_ref[...] = z - m - jnp.log(jnp.exp(z - m) + jnp.exp(other - m))


@jax.jit
def kernel(x, edge_index, dropout, W1, b1, W2, b2):
    n, f = x.shape
    e = edge_index.shape[1]
    dh = W1.shape[1]
    nc = W2.shape[1]

    n_pad = ((n + 127) // 128) * 128
    rt = n_pad // _NS

    chunks = (e + _K - 1) // _K
    gpw = (chunks + _NW - 1) // _NW
    gpw = ((gpw + _GRP - 1) // _GRP) * _GRP
    e_pad = _NW * gpw * _K
    src = jnp.concatenate(
        [edge_index[0], jnp.zeros((e_pad - e,), jnp.int32)])
    dst = jnp.concatenate(
        [edge_index[1], jnp.full((e_pad - e,), n_pad - 8, jnp.int32)])
    srcw = src.reshape(_NW * gpw, _K)
    dstw2 = dst.reshape(_NW * gpw // 2, 2 * _K)

    zeros_h = jnp.zeros((rt, dh), jnp.float32)
    ones_v = jnp.ones((2 * _K, dh), jnp.float32)

    eye4 = jnp.eye(4, dtype=jnp.float32)
    w1blk = jnp.kron(eye4, W1)
    w2big = jnp.kron(eye4, jnp.pad(W2, ((0, 0), (0, dh - nc))))
    b1t4 = jnp.tile(b1, 4).reshape(1, 128)
    b2t4 = jnp.tile(jnp.pad(b2, (0, dh - nc)), 4).reshape(1, 128)

    x4 = jnp.pad(x, ((0, n_pad - n), (0, 0))).reshape(n_pad // 4, 4 * f)

    r = 2176
    grid = n_pad // r
    r4 = r // 4

    degp = _make_degree(n_pad, dh, gpw)(dstw2, zeros_h, ones_v)
    dv4 = degp.reshape(_NC, n_pad // 4, 128)

    g1p4 = pl.pallas_call(
        _tc1_body,
        grid=(grid,),
        in_specs=[
            pl.BlockSpec((r4, 4 * f), lambda i: (i, 0)),
            pl.BlockSpec((4 * f, 128), lambda i: (0, 0)),
            pl.BlockSpec((_NC, r4, 128), lambda i: (0, i, 0)),
        ],
        out_specs=pl.BlockSpec((r4, 128), lambda i: (i, 0)),
        out_shape=jax.ShapeDtypeStruct((n_pad // 4, 128), jnp.float32),
    )(x4, w1blk, dv4)

    p1 = _make_edge_agg(n_pad, dh, gpw)(
        g1p4.reshape(n_pad, dh), srcw, dstw2, zeros_h)
    p1v = p1.reshape(_NC, n_pad // 4, 128)

    g2big = pl.pallas_call(
        _tc2_body,
        grid=(grid,),
        in_specs=[
            pl.BlockSpec((_NC, r4, 128), lambda i: (0, i, 0)),
            pl.BlockSpec((_NC, r4, 128), lambda i: (0, i, 0)),
            pl.BlockSpec((r4, 128), lambda i: (i, 0)),
            pl.BlockSpec((1, 128), lambda i: (0, 0)),
            pl.BlockSpec((128, 128), lambda i: (0, 0)),
        ],
        out_specs=pl.BlockSpec((r4, 128), lambda i: (i, 0)),
        out_shape=jax.ShapeDtypeStruct((n_pad // 4, 128), jnp.float32),
    )(dv4, p1v, g1p4, b1t4, w2big)

    p2 = _make_edge_agg(n_pad, dh, gpw)(
        g2big.reshape(n_pad, dh), srcw, dstw2, zeros_h)
    p2v = p2.reshape(_NC, n_pad // 4, 128)

    zfull = pl.pallas_call(
        _tc3_body,
        grid=(grid,),
        in_specs=[
            pl.BlockSpec((_NC, r4, 128), lambda i: (0, i, 0)),
            pl.BlockSpec((_NC, r4, 128), lambda i: (0, i, 0)),
            pl.BlockSpec((r4, 128), lambda i: (i, 0)),
            pl.BlockSpec((1, 128), lambda i: (0, 0)),
        ],
        out_specs=pl.BlockSpec((r4, 128), lambda i: (i, 0)),
        out_shape=jax.ShapeDtypeStruct((n_pad // 4, 128), jnp.float32),
    )(dv4, p2v, g2big, b2t4)

    c0 = lax.slice(zfull, (0, 0), (n_pad // 4, 128), (1, 32))
    c1 = lax.slice(zfull, (0, 1), (n_pad // 4, 128), (1, 32))
    return jnp.stack([c0, c1], axis=-1).reshape(n_pad, nc)[:n]


# speedup vs baseline: 1.1138x; 1.1138x over previous
import jax
import jax.numpy as jnp
from jax import lax
from jax.experimental import pallas as pl
from jax.experimental.pallas import tpu as pltpu
from jax.experimental.pallas import tpu_sc as plsc


_NC = 2
_NS = 16
_NW = _NC * _NS
_K = 128
_GRP = 28


def _make_edge_agg(n_pad, d, gpw):
    rt = n_pad // _NS
    spw = gpw // 2
    ngrp = gpw // _GRP
    sgrp = _GRP // 2
    mesh = plsc.VectorSubcoreMesh(core_axis_name="c", subcore_axis_name="s")

    def body(table, srcw, dstw2, zeros, out,
             src_v, dst_v, rows0, rows1, acc, ga0, ga1, sa0, sa1):
        cid = lax.axis_index("c")
        sid = lax.axis_index("s")
        w = cid * _NS + sid
        pltpu.sync_copy(zeros, acc.at[pl.ds(sid * rt, rt)])
        plsc.subcore_barrier()

        def gather(u, buf, sem):
            a = pltpu.async_copy(table.at[src_v.at[2 * u]],
                                 buf.at[pl.ds(0, _K)], sem)
            b = pltpu.async_copy(table.at[src_v.at[2 * u + 1]],
                                 buf.at[pl.ds(_K, _K)], sem)
            return a, b

        def scat(u, buf, sem):
            return pltpu.async_copy(buf, acc.at[dst_v.at[u]], sem, add=True)

        def group(g, carry):
            pltpu.sync_copy(srcw.at[pl.ds(w * gpw + g * _GRP, _GRP)], src_v)
            pltpu.sync_copy(dstw2.at[pl.ds(w * spw + g * sgrp, sgrp)], dst_v)
            a0, a1 = gather(0, rows0, ga0)
            b0, b1 = gather(1, rows1, ga1)

            def step(i, c2):
                u = 2 * i
                a0.wait()
                a1.wait()
                sA = scat(u, rows0, sa0)
                b0.wait()
                b1.wait()
                sB = scat(u + 1, rows1, sa1)
                sA.wait()
                gather(u + 2, rows0, ga0)
                sB.wait()
                gather(u + 3, rows1, ga1)
                return c2

            lax.fori_loop(0, sgrp // 2 - 1, step, 0)
            a0.wait()
            a1.wait()
            sA = scat(sgrp - 2, rows0, sa0)
            b0.wait()
            b1.wait()
            sB = scat(sgrp - 1, rows1, sa1)
            sA.wait()
            sB.wait()
            return carry

        lax.fori_loop(0, ngrp, group, 0)
        plsc.subcore_barrier()
        pltpu.sync_copy(acc.at[pl.ds(sid * rt, rt)],
                        out.at[cid, pl.ds(sid * rt, rt)])

    return pl.kernel(
        body,
        mesh=mesh,
        out_type=jax.ShapeDtypeStruct((_NC, n_pad, d), jnp.float32),
        compiler_params=pltpu.CompilerParams(use_tc_tiling_on_sc=False),
        scratch_types=[
            pltpu.VMEM((_GRP, _K), jnp.int32),
            pltpu.VMEM((sgrp, 2 * _K), jnp.int32),
            pltpu.VMEM((2 * _K, d), jnp.float32),
            pltpu.VMEM((2 * _K, d), jnp.float32),
            pltpu.VMEM_SHARED((n_pad, d), jnp.float32),
            pltpu.SemaphoreType.DMA,
            pltpu.SemaphoreType.DMA,
            pltpu.SemaphoreType.DMA,
            pltpu.SemaphoreType.DMA,
        ],
    )


def _make_degree(n_pad, d, gpw):
    rt = n_pad // _NS
    spw = gpw // 2
    ngrp = gpw // _GRP
    sgrp = _GRP // 2
    mesh = plsc.VectorSubcoreMesh(core_axis_name="c", subcore_axis_name="s")

    def body(dstw2, zeros, ones, out, dst_v, ones_v, acc, d0, d1):
        cid = lax.axis_index("c")
        sid = lax.axis_index("s")
        w = cid * _NS + sid
        pltpu.sync_copy(zeros, acc.at[pl.ds(sid * rt, rt)])
        pltpu.sync_copy(ones, ones_v)
        plsc.subcore_barrier()

        def group(g, carry):
            pltpu.sync_copy(dstw2.at[pl.ds(w * spw + g * sgrp, sgrp)], dst_v)
            sA = pltpu.async_copy(ones_v, acc.at[dst_v.at[0]], d0, add=True)
            sB = pltpu.async_copy(ones_v, acc.at[dst_v.at[1]], d1, add=True)

            def step(i, c2):
                u = 2 * i
                sA.wait()
                pltpu.async_copy(ones_v, acc.at[dst_v.at[u + 2]], d0, add=True)
                sB.wait()
                pltpu.async_copy(ones_v, acc.at[dst_v.at[u + 3]], d1, add=True)
                return c2

            lax.fori_loop(0, sgrp // 2 - 1, step, 0)
            sA.wait()
            sB.wait()
            return carry

        lax.fori_loop(0, ngrp, group, 0)
        plsc.subcore_barrier()
        pltpu.sync_copy(acc.at[pl.ds(sid * rt, rt)],
                        out.at[cid, pl.ds(sid * rt, rt)])

    return pl.kernel(
        body,
        mesh=mesh,
        out_type=jax.ShapeDtypeStruct((_NC, n_pad, d), jnp.float32),
        compiler_params=pltpu.CompilerParams(use_tc_tiling_on_sc=False),
        scratch_types=[
            pltpu.VMEM((sgrp, 2 * _K), jnp.int32),
            pltpu.VMEM((2 * _K, d), jnp.float32),
            pltpu.VMEM_SHARED((n_pad, d), jnp.float32),
            pltpu.SemaphoreType.DMA,
            pltpu.SemaphoreType.DMA,
        ],
    )


def _tc1_body(x4_ref, w1blk_ref, dv4_ref, g1_ref):
    dinv4 = lax.rsqrt(1.0 + dv4_ref[0] + dv4_ref[1])
    h4 = jnp.dot(x4_ref[...], w1blk_ref[...],
                 preferred_element_type=jnp.float32)
    g1_ref[...] = dinv4 * h4


def _tc2_body(dv4_ref, p_ref, g1_ref, b1t4_ref, w2big_ref, g2_ref):
    dinv4 = lax.rsqrt(1.0 + dv4_ref[0] + dv4_ref[1])
    agg = p_ref[0] + p_ref[1] + g1_ref[...]
    out1 = jnp.maximum(dinv4 * agg + b1t4_ref[...], 0.0)
    g2_ref[...] = jnp.dot(dinv4 * out1, w2big_ref[...],
                          preferred_element_type=jnp.float32)


def _tc3_body(dv4_ref, p_ref, g2_ref, b2t4_ref, sel_ref, o_ref):
    r4 = g2_ref.shape[0]
    dinv4 = lax.rsqrt(1.0 + dv4_ref[0] + dv4_ref[1])
    z = dinv4 * (p_ref[0] + p_ref[1] + g2_ref[...]) + b2t4_ref[...]
    lane = lax.broadcasted_iota(jnp.int32, (r4, 128), 1)
    other = jnp.where(lane % 32 == 0, jnp.roll(z, -1, axis=1),
                      jnp.roll(z, 1, axis=1))
    m = jnp.maximum(z, other)
    logp = z - m - jnp.log(jnp.exp(z - m) + jnp.exp(other - m))
    o_ref[...] = jnp.dot(logp, sel_ref[...],
                         preferred_element_type=jnp.float32)


@jax.jit
def kernel(x, edge_index, dropout, W1, b1, W2, b2):
    n, f = x.shape
    e = edge_index.shape[1]
    dh = W1.shape[1]
    nc = W2.shape[1]

    n_pad = ((n + 127) // 128) * 128
    rt = n_pad // _NS

    chunks = (e + _K - 1) // _K
    gpw = (chunks + _NW - 1) // _NW
    gpw = ((gpw + _GRP - 1) // _GRP) * _GRP
    e_pad = _NW * gpw * _K
    src = jnp.concatenate(
        [edge_index[0], jnp.zeros((e_pad - e,), jnp.int32)])
    dst = jnp.concatenate(
        [edge_index[1], jnp.full((e_pad - e,), n_pad - 8, jnp.int32)])
    srcw = src.reshape(_NW * gpw, _K)
    dstw2 = dst.reshape(_NW * gpw // 2, 2 * _K)

    zeros_h = jnp.zeros((rt, dh), jnp.float32)
    ones_v = jnp.ones((2 * _K, dh), jnp.float32)

    eye4 = jnp.eye(4, dtype=jnp.float32)
    w1blk = jnp.kron(eye4, W1)
    w2big = jnp.kron(eye4, jnp.pad(W2, ((0, 0), (0, dh - nc))))
    b1t4 = jnp.tile(b1, 4).reshape(1, 128)
    b2t4 = jnp.tile(jnp.pad(b2, (0, dh - nc)), 4).reshape(1, 128)
    sel = jnp.kron(eye4, jnp.pad(jnp.eye(nc, dtype=jnp.float32),
                                 ((0, dh - nc), (0, 0))))

    x4 = jnp.pad(x, ((0, n_pad - n), (0, 0))).reshape(n_pad // 4, 4 * f)

    r = 2176
    grid = n_pad // r
    r4 = r // 4

    degp = _make_degree(n_pad, dh, gpw)(dstw2, zeros_h, ones_v)
    dv4 = degp.reshape(_NC, n_pad // 4, 128)

    g1p4 = pl.pallas_call(
        _tc1_body,
        grid=(grid,),
        in_specs=[
            pl.BlockSpec((r4, 4 * f), lambda i: (i, 0)),
            pl.BlockSpec((4 * f, 128), lambda i: (0, 0)),
            pl.BlockSpec((_NC, r4, 128), lambda i: (0, i, 0)),
        ],
        out_specs=pl.BlockSpec((r4, 128), lambda i: (i, 0)),
        out_shape=jax.ShapeDtypeStruct((n_pad // 4, 128), jnp.float32),
    )(x4, w1blk, dv4)

    p1 = _make_edge_agg(n_pad, dh, gpw)(
        g1p4.reshape(n_pad, dh), srcw, dstw2, zeros_h)
    p1v = p1.reshape(_NC, n_pad // 4, 128)

    g2big = pl.pallas_call(
        _tc2_body,
        grid=(grid,),
        in_specs=[
            pl.BlockSpec((_NC, r4, 128), lambda i: (0, i, 0)),
            pl.BlockSpec((_NC, r4, 128), lambda i: (0, i, 0)),
            pl.BlockSpec((r4, 128), lambda i: (i, 0)),
            pl.BlockSpec((1, 128), lambda i: (0, 0)),
            pl.BlockSpec((128, 128), lambda i: (0, 0)),
        ],
        out_specs=pl.BlockSpec((r4, 128), lambda i: (i, 0)),
        out_shape=jax.ShapeDtypeStruct((n_pad // 4, 128), jnp.float32),
    )(dv4, p1v, g1p4, b1t4, w2big)

    p2 = _make_edge_agg(n_pad, dh, gpw)(
        g2big.reshape(n_pad, dh), srcw, dstw2, zeros_h)
    p2v = p2.reshape(_NC, n_pad // 4, 128)

    zsel = pl.pallas_call(
        _tc3_body,
        grid=(grid,),
        in_specs=[
            pl.BlockSpec((_NC, r4, 128), lambda i: (0, i, 0)),
            pl.BlockSpec((_NC, r4, 128), lambda i: (0, i, 0)),
            pl.BlockSpec((r4, 128), lambda i: (i, 0)),
            pl.BlockSpec((1, 128), lambda i: (0, 0)),
            pl.BlockSpec((128, 4 * nc), lambda i: (0, 0)),
        ],
        out_specs=pl.BlockSpec((r4, 4 * nc), lambda i: (i, 0)),
        out_shape=jax.ShapeDtypeStruct((n_pad // 4, 4 * nc), jnp.float32),
    )(dv4, p2v, g2big, b2t4, sel)

    return zsel.reshape(n_pad, nc)[:n]


# speedup vs baseline: 1.2277x; 1.1023x over previous
import jax
import jax.numpy as jnp
from jax import lax
from jax.experimental import pallas as pl
from jax.experimental.pallas import tpu as pltpu
from jax.experimental.pallas import tpu_sc as plsc


_NC = 2
_NS = 16
_NW = _NC * _NS
_K = 128
_GRP = 28


def _make_edge_agg(n_pad, d, gpw):
    rt = n_pad // _NS
    spw = gpw // 2
    ngrp = gpw // _GRP
    sgrp = _GRP // 2
    mesh = plsc.VectorSubcoreMesh(core_axis_name="c", subcore_axis_name="s")

    def body(table, srcw, dstw2, zeros, out,
             src_v, dst_v, rows0, rows1, acc, ga0, ga1, sa0, sa1):
        cid = lax.axis_index("c")
        sid = lax.axis_index("s")
        w = cid * _NS + sid
        pltpu.sync_copy(zeros, acc.at[pl.ds(sid * rt, rt)])
        plsc.subcore_barrier()

        def gather(u, buf, sem):
            a = pltpu.async_copy(table.at[src_v.at[2 * u]],
                                 buf.at[pl.ds(0, _K)], sem)
            b = pltpu.async_copy(table.at[src_v.at[2 * u + 1]],
                                 buf.at[pl.ds(_K, _K)], sem)
            return a, b

        def scat(u, buf, sem):
            return pltpu.async_copy(buf, acc.at[dst_v.at[u]], sem, add=True)

        def group(g, carry):
            pltpu.sync_copy(srcw.at[pl.ds(w * gpw + g * _GRP, _GRP)], src_v)
            pltpu.sync_copy(dstw2.at[pl.ds(w * spw + g * sgrp, sgrp)], dst_v)
            a0, a1 = gather(0, rows0, ga0)
            b0, b1 = gather(1, rows1, ga1)

            def step(i, c2):
                u = 2 * i
                a0.wait()
                a1.wait()
                sA = scat(u, rows0, sa0)
                b0.wait()
                b1.wait()
                sB = scat(u + 1, rows1, sa1)
                sA.wait()
                gather(u + 2, rows0, ga0)
                sB.wait()
                gather(u + 3, rows1, ga1)
                return c2

            lax.fori_loop(0, sgrp // 2 - 1, step, 0)
            a0.wait()
            a1.wait()
            sA = scat(sgrp - 2, rows0, sa0)
            b0.wait()
            b1.wait()
            sB = scat(sgrp - 1, rows1, sa1)
            sA.wait()
            sB.wait()
            return carry

        lax.fori_loop(0, ngrp, group, 0)
        plsc.subcore_barrier()
        pltpu.sync_copy(acc.at[pl.ds(sid * rt, rt)],
                        out.at[cid, pl.ds(sid * rt, rt)])

    return pl.kernel(
        body,
        mesh=mesh,
        out_type=jax.ShapeDtypeStruct((_NC, n_pad, d), jnp.float32),
        compiler_params=pltpu.CompilerParams(use_tc_tiling_on_sc=False),
        scratch_types=[
            pltpu.VMEM((_GRP, _K), jnp.int32),
            pltpu.VMEM((sgrp, 2 * _K), jnp.int32),
            pltpu.VMEM((2 * _K, d), jnp.float32),
            pltpu.VMEM((2 * _K, d), jnp.float32),
            pltpu.VMEM_SHARED((n_pad, d), jnp.float32),
            pltpu.SemaphoreType.DMA,
            pltpu.SemaphoreType.DMA,
            pltpu.SemaphoreType.DMA,
            pltpu.SemaphoreType.DMA,
        ],
    )


def _make_degree(n_pad, d, gpw):
    rt = n_pad // _NS
    spw = gpw // 2
    ngrp = gpw // _GRP
    sgrp = _GRP // 2
    mesh = plsc.VectorSubcoreMesh(core_axis_name="c", subcore_axis_name="s")

    def body(dstw2, zeros, ones, out, dst_v, ones_v, acc, d0, d1):
        cid = lax.axis_index("c")
        sid = lax.axis_index("s")
        w = cid * _NS + sid
        pltpu.sync_copy(zeros, acc.at[pl.ds(sid * rt, rt)])
        pltpu.sync_copy(ones, ones_v)
        plsc.subcore_barrier()

        def group(g, carry):
            pltpu.sync_copy(dstw2.at[pl.ds(w * spw + g * sgrp, sgrp)], dst_v)
            sA = pltpu.async_copy(ones_v, acc.at[dst_v.at[0]], d0, add=True)
            sB = pltpu.async_copy(ones_v, acc.at[dst_v.at[1]], d1, add=True)

            def step(i, c2):
                u = 2 * i
                sA.wait()
                pltpu.async_copy(ones_v, acc.at[dst_v.at[u + 2]], d0, add=True)
                sB.wait()
                pltpu.async_copy(ones_v, acc.at[dst_v.at[u + 3]], d1, add=True)
                return c2

            lax.fori_loop(0, sgrp // 2 - 1, step, 0)
            sA.wait()
            sB.wait()
            return carry

        lax.fori_loop(0, ngrp, group, 0)
        plsc.subcore_barrier()
        pltpu.sync_copy(acc.at[pl.ds(sid * rt, rt)],
                        out.at[cid, pl.ds(sid * rt, rt)])

    return pl.kernel(
        body,
        mesh=mesh,
        out_type=jax.ShapeDtypeStruct((_NC, n_pad, d), jnp.float32),
        compiler_params=pltpu.CompilerParams(use_tc_tiling_on_sc=False),
        scratch_types=[
            pltpu.VMEM((sgrp, 2 * _K), jnp.int32),
            pltpu.VMEM((2 * _K, d), jnp.float32),
            pltpu.VMEM_SHARED((n_pad, d), jnp.float32),
            pltpu.SemaphoreType.DMA,
            pltpu.SemaphoreType.DMA,
        ],
    )


def _tc1_body(x4_ref, w1blk_ref, dv4_ref, g1_ref):
    dinv4 = lax.rsqrt(1.0 + dv4_ref[0] + dv4_ref[1])
    h4 = jnp.dot(x4_ref[...], w1blk_ref[...],
                 preferred_element_type=jnp.float32)
    g1_ref[...] = dinv4 * h4


def _tc2_body(dv4_ref, p_ref, g1_ref, b1t4_ref, w2sel_ref, g2_ref):
    dinv4 = lax.rsqrt(1.0 + dv4_ref[0] + dv4_ref[1])
    agg = p_ref[0] + p_ref[1] + g1_ref[...]
    out1 = jnp.maximum(dinv4 * agg + b1t4_ref[...], 0.0)
    g2_ref[...] = jnp.dot(dinv4 * out1, w2sel_ref[...],
                          preferred_element_type=jnp.float32)


def _tc3_body(dv4_ref, p_ref, g2_ref, b2t4_ref, sel_ref, xp_ref, o_ref):
    r4 = g2_ref.shape[0]
    dinv4 = lax.rsqrt(1.0 + dv4_ref[0] + dv4_ref[1])
    zc = p_ref[0] + p_ref[1] + g2_ref[...]
    zbig = jnp.dot(zc, xp_ref[...], preferred_element_type=jnp.float32)
    z = dinv4 * zbig + b2t4_ref[...]
    lane = lax.broadcasted_iota(jnp.int32, (r4, 128), 1)
    other = jnp.where(lane % 32 == 0, jnp.roll(z, -1, axis=1),
                      jnp.roll(z, 1, axis=1))
    m = jnp.maximum(z, other)
    logp = z - m - jnp.log(jnp.exp(z - m) + jnp.exp(other - m))
    o_ref[...] = jnp.dot(logp, sel_ref[...],
                         preferred_element_type=jnp.float32)


@jax.jit
def kernel(x, edge_index, dropout, W1, b1, W2, b2):
    n, f = x.shape
    e = edge_index.shape[1]
    dh = W1.shape[1]
    nc = W2.shape[1]

    n_pad = ((n + 127) // 128) * 128
    rt = n_pad // _NS

    chunks = (e + _K - 1) // _K
    gpw = (chunks + _NW - 1) // _NW
    gpw = ((gpw + _GRP - 1) // _GRP) * _GRP
    e_pad = _NW * gpw * _K
    src = jnp.concatenate(
        [edge_index[0], jnp.zeros((e_pad - e,), jnp.int32)])
    dst = jnp.concatenate(
        [edge_index[1], jnp.full((e_pad - e,), n_pad - 8, jnp.int32)])
    srcw = src.reshape(_NW * gpw, _K)
    dstw2 = dst.reshape(_NW * gpw // 2, 2 * _K)

    zeros_h = jnp.zeros((rt, dh), jnp.float32)
    zeros_p = jnp.zeros((rt, 8), jnp.float32)
    ones_v = jnp.ones((2 * _K, dh), jnp.float32)

    eye4 = jnp.eye(4, dtype=jnp.float32)
    w1blk = jnp.kron(eye4, W1)
    dp = 8
    w2sel = jnp.kron(eye4, jnp.pad(W2, ((0, 0), (0, dp - nc))))
    xp = jnp.kron(eye4, jnp.pad(jnp.eye(dp, dtype=jnp.float32),
                                ((0, 0), (0, dh - dp))))
    b1t4 = jnp.tile(b1, 4).reshape(1, 128)
    b2t4 = jnp.tile(jnp.pad(b2, (0, dh - nc)), 4).reshape(1, 128)
    sel = jnp.kron(eye4, jnp.pad(jnp.eye(nc, dtype=jnp.float32),
                                 ((0, dh - nc), (0, 0))))

    x4 = jnp.pad(x, ((0, n_pad - n), (0, 0))).reshape(n_pad // 4, 4 * f)

    r = 2176
    grid = n_pad // r
    r4 = r // 4

    degp = _make_degree(n_pad, dh, gpw)(dstw2, zeros_h, ones_v)
    dv4 = degp.reshape(_NC, n_pad // 4, 128)

    g1p4 = pl.pallas_call(
        _tc1_body,
        grid=(grid,),
        in_specs=[
            pl.BlockSpec((r4, 4 * f), lambda i: (i, 0)),
            pl.BlockSpec((4 * f, 128), lambda i: (0, 0)),
            pl.BlockSpec((_NC, r4, 128), lambda i: (0, i, 0)),
        ],
        out_specs=pl.BlockSpec((r4, 128), lambda i: (i, 0)),
        out_shape=jax.ShapeDtypeStruct((n_pad // 4, 128), jnp.float32),
    )(x4, w1blk, dv4)

    p1 = _make_edge_agg(n_pad, dh, gpw)(
        g1p4.reshape(n_pad, dh), srcw, dstw2, zeros_h)
    p1v = p1.reshape(_NC, n_pad // 4, 128)

    g2sel = pl.pallas_call(
        _tc2_body,
        grid=(grid,),
        in_specs=[
            pl.BlockSpec((_NC, r4, 128), lambda i: (0, i, 0)),
            pl.BlockSpec((_NC, r4, 128), lambda i: (0, i, 0)),
            pl.BlockSpec((r4, 128), lambda i: (i, 0)),
            pl.BlockSpec((1, 128), lambda i: (0, 0)),
            pl.BlockSpec((128, 32), lambda i: (0, 0)),
        ],
        out_specs=pl.BlockSpec((r4, 32), lambda i: (i, 0)),
        out_shape=jax.ShapeDtypeStruct((n_pad // 4, 32), jnp.float32),
    )(dv4, p1v, g1p4, b1t4, w2sel)

    p2 = _make_edge_agg(n_pad, 8, gpw)(
        g2sel.reshape(n_pad, 8), srcw, dstw2, zeros_p)
    p2v = p2.reshape(_NC, n_pad // 4, 32)

    zsel = pl.pallas_call(
        _tc3_body,
        grid=(grid,),
        in_specs=[
            pl.BlockSpec((_NC, r4, 128), lambda i: (0, i, 0)),
            pl.BlockSpec((_NC, r4, 32), lambda i: (0, i, 0)),
            pl.BlockSpec((r4, 32), lambda i: (i, 0)),
            pl.BlockSpec((1, 128), lambda i: (0, 0)),
            pl.BlockSpec((128, 4 * nc), lambda i: (0, 0)),
            pl.BlockSpec((32, 128), lambda i: (0, 0)),
        ],
        out_specs=pl.BlockSpec((r4, 4 * nc), lambda i: (i, 0)),
        out_shape=jax.ShapeDtypeStruct((n_pad // 4, 4 * nc), jnp.float32),
    )(dv4, p2v, g2sel, b2t4, sel, xp)

    return zsel.reshape(n_pad, nc)[:n]


# speedup vs baseline: 1.3094x; 1.0665x over previous
import jax
import jax.numpy as jnp
from jax import lax
from jax.experimental import pallas as pl
from jax.experimental.pallas import tpu as pltpu
from jax.experimental.pallas import tpu_sc as plsc


_NC = 2
_NS = 16
_NW = _NC * _NS
_K = 128
_GRP = 28


def _make_edge_agg(n_pad, d, gpw):
    rt = n_pad // _NS
    spw = gpw // 2
    ngrp = gpw // _GRP
    sgrp = _GRP // 2
    mesh = plsc.VectorSubcoreMesh(core_axis_name="c", subcore_axis_name="s")

    def body(table, srcw, dstw2, zeros, out,
             src_v, dst_v, rows0, rows1, acc, ga0, ga1, sa0, sa1):
        cid = lax.axis_index("c")
        sid = lax.axis_index("s")
        w = cid * _NS + sid
        pltpu.sync_copy(zeros, acc.at[pl.ds(sid * rt, rt)])
        plsc.subcore_barrier()

        def gather(u, buf, sem):
            a = pltpu.async_copy(table.at[src_v.at[2 * u]],
                                 buf.at[pl.ds(0, _K)], sem)
            b = pltpu.async_copy(table.at[src_v.at[2 * u + 1]],
                                 buf.at[pl.ds(_K, _K)], sem)
            return a, b

        def scat(u, buf, sem):
            return pltpu.async_copy(buf, acc.at[dst_v.at[u]], sem, add=True)

        def group(g, carry):
            pltpu.sync_copy(srcw.at[pl.ds(w * gpw + g * _GRP, _GRP)], src_v)
            pltpu.sync_copy(dstw2.at[pl.ds(w * spw + g * sgrp, sgrp)], dst_v)
            a0, a1 = gather(0, rows0, ga0)
            b0, b1 = gather(1, rows1, ga1)

            def step(i, c2):
                u = 2 * i
                a0.wait()
                a1.wait()
                sA = scat(u, rows0, sa0)
                b0.wait()
                b1.wait()
                sB = scat(u + 1, rows1, sa1)
                sA.wait()
                gather(u + 2, rows0, ga0)
                sB.wait()
                gather(u + 3, rows1, ga1)
                return c2

            lax.fori_loop(0, sgrp // 2 - 1, step, 0)
            a0.wait()
            a1.wait()
            sA = scat(sgrp - 2, rows0, sa0)
            b0.wait()
            b1.wait()
            sB = scat(sgrp - 1, rows1, sa1)
            sA.wait()
            sB.wait()
            return carry

        lax.fori_loop(0, ngrp, group, 0)
        plsc.subcore_barrier()
        pltpu.sync_copy(acc.at[pl.ds(sid * rt, rt)],
                        out.at[cid, pl.ds(sid * rt, rt)])

    return pl.kernel(
        body,
        mesh=mesh,
        out_type=jax.ShapeDtypeStruct((_NC, n_pad, d), jnp.float32),
        compiler_params=pltpu.CompilerParams(use_tc_tiling_on_sc=False),
        scratch_types=[
            pltpu.VMEM((_GRP, _K), jnp.int32),
            pltpu.VMEM((sgrp, 2 * _K), jnp.int32),
            pltpu.VMEM((2 * _K, d), jnp.float32),
            pltpu.VMEM((2 * _K, d), jnp.float32),
            pltpu.VMEM_SHARED((n_pad, d), jnp.float32),
            pltpu.SemaphoreType.DMA,
            pltpu.SemaphoreType.DMA,
            pltpu.SemaphoreType.DMA,
            pltpu.SemaphoreType.DMA,
        ],
    )


def _make_edge_agg8(n_pad, gpw):
    d = 8
    rt = n_pad // _NS
    spw = gpw // 2
    nq = spw // 4 - 1
    mesh = plsc.VectorSubcoreMesh(core_axis_name="c", subcore_axis_name="s")

    def body(table, srcw, dstw2, zeros, out,
             src_v, dst_v, r0, r1, r2, r3, acc,
             g0, g1, g2, g3, s0, s1, s2, s3):
        cid = lax.axis_index("c")
        sid = lax.axis_index("s")
        w = cid * _NS + sid
        pltpu.sync_copy(zeros, acc.at[pl.ds(sid * rt, rt)])
        pltpu.sync_copy(srcw.at[pl.ds(w * gpw, gpw)], src_v)
        pltpu.sync_copy(dstw2.at[pl.ds(w * spw, spw)], dst_v)
        plsc.subcore_barrier()

        bufs = (r0, r1, r2, r3)
        gsem = (g0, g1, g2, g3)
        ssem = (s0, s1, s2, s3)

        def gather(u, buf, sem):
            a = pltpu.async_copy(table.at[src_v.at[2 * u]],
                                 buf.at[pl.ds(0, _K)], sem)
            b = pltpu.async_copy(table.at[src_v.at[2 * u + 1]],
                                 buf.at[pl.ds(_K, _K)], sem)
            return a, b

        pro = [gather(b, bufs[b], gsem[b]) for b in range(4)]

        def quad(i, c):
            u0 = 4 * i
            for b in range(4):
                pro[b][0].wait()
                pro[b][1].wait()
                sc = pltpu.async_copy(bufs[b], acc.at[dst_v.at[u0 + b]],
                                      ssem[b], add=True)
                sc.wait()
                gather(u0 + b + 4, bufs[b], gsem[b])
            return c

        lax.fori_loop(0, nq, quad, 0)
        base = 4 * nq
        for b in range(4):
            pro[b][0].wait()
            pro[b][1].wait()
            pltpu.sync_copy(bufs[b], acc.at[dst_v.at[base + b]], add=True)
        for b in range(spw - base - 4):
            u = base + 4 + b
            a, bb = gather(u, bufs[b], gsem[b])
            a.wait()
            bb.wait()
            pltpu.sync_copy(bufs[b], acc.at[dst_v.at[u]], add=True)
        plsc.subcore_barrier()
        pltpu.sync_copy(acc.at[pl.ds(sid * rt, rt)],
                        out.at[cid, pl.ds(sid * rt, rt)])

    kern = pl.kernel(
        body,
        mesh=mesh,
        out_type=jax.ShapeDtypeStruct((_NC, n_pad, d), jnp.float32),
        compiler_params=pltpu.CompilerParams(use_tc_tiling_on_sc=False),
        scratch_types=[
            pltpu.VMEM((gpw, _K), jnp.int32),
            pltpu.VMEM((spw, 2 * _K), jnp.int32),
            pltpu.VMEM((2 * _K, d), jnp.float32),
            pltpu.VMEM((2 * _K, d), jnp.float32),
            pltpu.VMEM((2 * _K, d), jnp.float32),
            pltpu.VMEM((2 * _K, d), jnp.float32),
            pltpu.VMEM_SHARED((n_pad, d), jnp.float32),
        ] + [pltpu.SemaphoreType.DMA] * 8,
    )
    return kern


def _make_degree(n_pad, d, gpw):
    rt = n_pad // _NS
    spw = gpw // 2
    ngrp = gpw // _GRP
    sgrp = _GRP // 2
    mesh = plsc.VectorSubcoreMesh(core_axis_name="c", subcore_axis_name="s")

    def body(dstw2, zeros, ones, out, dst_v, ones_v, acc, d0, d1, d2, d3):
        cid = lax.axis_index("c")
        sid = lax.axis_index("s")
        w = cid * _NS + sid
        pltpu.sync_copy(zeros, acc.at[pl.ds(sid * rt, rt)])
        pltpu.sync_copy(ones, ones_v)
        plsc.subcore_barrier()

        sems = (d0, d1, d2, d3)

        def group(g, carry):
            pltpu.sync_copy(dstw2.at[pl.ds(w * spw + g * sgrp, sgrp)], dst_v)
            pro = [pltpu.async_copy(ones_v, acc.at[dst_v.at[b]], sems[b],
                                    add=True) for b in range(4)]
            nq = sgrp // 4 - 1

            def quad(i, c2):
                u0 = 4 * i
                for b in range(4):
                    pro[b].wait()
                    pltpu.async_copy(ones_v, acc.at[dst_v.at[u0 + b + 4]],
                                     sems[b], add=True)
                return c2

            lax.fori_loop(0, nq, quad, 0)
            base = 4 * nq + 4
            for b in range(4):
                pro[b].wait()
            for b in range(sgrp - base):
                pltpu.sync_copy(ones_v, acc.at[dst_v.at[base + b]], add=True)
            return carry

        lax.fori_loop(0, ngrp, group, 0)
        plsc.subcore_barrier()
        pltpu.sync_copy(acc.at[pl.ds(sid * rt, rt)],
                        out.at[cid, pl.ds(sid * rt, rt)])

    return pl.kernel(
        body,
        mesh=mesh,
        out_type=jax.ShapeDtypeStruct((_NC, n_pad, d), jnp.float32),
        compiler_params=pltpu.CompilerParams(use_tc_tiling_on_sc=False),
        scratch_types=[
            pltpu.VMEM((sgrp, 2 * _K), jnp.int32),
            pltpu.VMEM((2 * _K, d), jnp.float32),
            pltpu.VMEM_SHARED((n_pad, d), jnp.float32),
        ] + [pltpu.SemaphoreType.DMA] * 4,
    )


def _tc1_body(x4_ref, w1blk_ref, dv4_ref, g1_ref):
    dinv4 = lax.rsqrt(1.0 + dv4_ref[0] + dv4_ref[1])
    h4 = jnp.dot(x4_ref[...], w1blk_ref[...],
                 preferred_element_type=jnp.float32)
    g1_ref[...] = dinv4 * h4


def _tc2_body(dv4_ref, p_ref, g1_ref, b1t4_ref, w2sel_ref, g2_ref):
    dinv4 = lax.rsqrt(1.0 + dv4_ref[0] + dv4_ref[1])
    agg = p_ref[0] + p_ref[1] + g1_ref[...]
    out1 = jnp.maximum(dinv4 * agg + b1t4_ref[...], 0.0)
    g2_ref[...] = jnp.dot(dinv4 * out1, w2sel_ref[...],
                          preferred_element_type=jnp.float32)


def _tc3_body(dv4_ref, p_ref, g2_ref, b2t4_ref, sel_ref, xp_ref, o_ref):
    r4 = g2_ref.shape[0]
    dinv4 = lax.rsqrt(1.0 + dv4_ref[0] + dv4_ref[1])
    zc = p_ref[0] + p_ref[1] + g2_ref[...]
    zbig = jnp.dot(zc, xp_ref[...], preferred_element_type=jnp.float32)
    z = dinv4 * zbig + b2t4_ref[...]
    lane = lax.broadcasted_iota(jnp.int32, (r4, 128), 1)
    other = jnp.where(lane % 32 == 0, jnp.roll(z, -1, axis=1),
                      jnp.roll(z, 1, axis=1))
    m = jnp.maximum(z, other)
    logp = z - m - jnp.log(jnp.exp(z - m) + jnp.exp(other - m))
    o_ref[...] = jnp.dot(logp, sel_ref[...],
                         preferred_element_type=jnp.float32)


@jax.jit
def kernel(x, edge_index, dropout, W1, b1, W2, b2):
    n, f = x.shape
    e = edge_index.shape[1]
    dh = W1.shape[1]
    nc = W2.shape[1]

    n_pad = ((n + 127) // 128) * 128
    rt = n_pad // _NS

    chunks = (e + _K - 1) // _K
    gpw = (chunks + _NW - 1) // _NW
    gpw = ((gpw + _GRP - 1) // _GRP) * _GRP
    e_pad = _NW * gpw * _K
    src = jnp.concatenate(
        [edge_index[0], jnp.zeros((e_pad - e,), jnp.int32)])
    dst = jnp.concatenate(
        [edge_index[1], jnp.full((e_pad - e,), n_pad - 8, jnp.int32)])
    srcw = src.reshape(_NW * gpw, _K)
    dstw2 = dst.reshape(_NW * gpw // 2, 2 * _K)

    zeros_h = jnp.zeros((rt, dh), jnp.float32)
    zeros_p = jnp.zeros((rt, 8), jnp.float32)
    ones_v = jnp.ones((2 * _K, dh), jnp.float32)

    eye4 = jnp.eye(4, dtype=jnp.float32)
    w1blk = jnp.kron(eye4, W1)
    dp = 8
    w2sel = jnp.kron(eye4, jnp.pad(W2, ((0, 0), (0, dp - nc))))
    xp = jnp.kron(eye4, jnp.pad(jnp.eye(dp, dtype=jnp.float32),
                                ((0, 0), (0, dh - dp))))
    b1t4 = jnp.tile(b1, 4).reshape(1, 128)
    b2t4 = jnp.tile(jnp.pad(b2, (0, dh - nc)), 4).reshape(1, 128)
    sel = jnp.kron(eye4, jnp.pad(jnp.eye(nc, dtype=jnp.float32),
                                 ((0, dh - nc), (0, 0))))

    x4 = jnp.pad(x, ((0, n_pad - n), (0, 0))).reshape(n_pad // 4, 4 * f)

    r = 2176
    grid = n_pad // r
    r4 = r // 4

    degp = _make_degree(n_pad, dh, gpw)(dstw2, zeros_h, ones_v)
    dv4 = degp.reshape(_NC, n_pad // 4, 128)

    g1p4 = pl.pallas_call(
        _tc1_body,
        grid=(grid,),
        in_specs=[
            pl.BlockSpec((r4, 4 * f), lambda i: (i, 0)),
            pl.BlockSpec((4 * f, 128), lambda i: (0, 0)),
            pl.BlockSpec((_NC, r4, 128), lambda i: (0, i, 0)),
        ],
        out_specs=pl.BlockSpec((r4, 128), lambda i: (i, 0)),
        out_shape=jax.ShapeDtypeStruct((n_pad // 4, 128), jnp.float32),
    )(x4, w1blk, dv4)

    p1 = _make_edge_agg(n_pad, dh, gpw)(
        g1p4.reshape(n_pad, dh), srcw, dstw2, zeros_h)
    p1v = p1.reshape(_NC, n_pad // 4, 128)

    g2sel = pl.pallas_call(
        _tc2_body,
        grid=(grid,),
        in_specs=[
            pl.BlockSpec((_NC, r4, 128), lambda i: (0, i, 0)),
            pl.BlockSpec((_NC, r4, 128), lambda i: (0, i, 0)),
            pl.BlockSpec((r4, 128), lambda i: (i, 0)),
            pl.BlockSpec((1, 128), lambda i: (0, 0)),
            pl.BlockSpec((128, 32), lambda i: (0, 0)),
        ],
        out_specs=pl.BlockSpec((r4, 32), lambda i: (i, 0)),
        out_shape=jax.ShapeDtypeStruct((n_pad // 4, 32), jnp.float32),
    )(dv4, p1v, g1p4, b1t4, w2sel)

    p2 = _make_edge_agg8(n_pad, gpw)(
        g2sel.reshape(n_pad, 8), srcw, dstw2, zeros_p)
    p2v = p2.reshape(_NC, n_pad // 4, 32)

    zsel = pl.pallas_call(
        _tc3_body,
        grid=(grid,),
        in_specs=[
            pl.BlockSpec((_NC, r4, 128), lambda i: (0, i, 0)),
            pl.BlockSpec((_NC, r4, 32), lambda i: (0, i, 0)),
            pl.BlockSpec((r4, 32), lambda i: (i, 0)),
            pl.BlockSpec((1, 128), lambda i: (0, 0)),
            pl.BlockSpec((128, 4 * nc), lambda i: (0, 0)),
            pl.BlockSpec((32, 128), lambda i: (0, 0)),
        ],
        out_specs=pl.BlockSpec((r4, 4 * nc), lambda i: (i, 0)),
        out_shape=jax.ShapeDtypeStruct((n_pad // 4, 4 * nc), jnp.float32),
    )(dv4, p2v, g2sel, b2t4, sel, xp)

    return zsel.reshape(n_pad, nc)[:n]
